# Initial kernel scaffold; baseline (speedup 1.0000x reference)
import jax
import jax.numpy as jnp
from jax.experimental import pallas as pl


def kernel(x, table):
    raise NotImplementedError("write your pallas kernel here")


# speedup vs baseline: 4.1371x; 4.1371x over previous
import functools

import jax
import jax.numpy as jnp
from jax import lax
from jax.experimental import pallas as pl
from jax.experimental.pallas import tpu as pltpu
from jax.experimental.pallas import tpu_sc as plsc

_info = plsc.get_sparse_core_info()
_NC = _info.num_cores
_NS = _info.num_subcores
_NW = _NC * _NS
_L = _info.num_lanes

_ROWS = 10
_W = 10
_CHUNK = 2048


def _sc_lookup(n_idx: int):
    n_per_w = n_idx // _NW
    n_chunks = n_per_w // _CHUNK
    assert n_per_w % _CHUNK == 0 and _CHUNK % _L == 0

    mesh = plsc.VectorSubcoreMesh(core_axis_name="c", subcore_axis_name="s")

    @functools.partial(
        pl.kernel,
        mesh=mesh,
        out_type=jax.ShapeDtypeStruct((n_idx * _W,), jnp.int32),
        compiler_params=pltpu.CompilerParams(needs_layout_passes=False),
        scratch_types=[
            pltpu.VMEM((_ROWS * _W,), jnp.int32),
            pltpu.VMEM((_CHUNK,), jnp.int32),
            pltpu.VMEM((_CHUNK * _W,), jnp.int32),
        ],
    )
    def body(table_hbm, idx_hbm, out_hbm, table_v, idx_v, out_v):
        wid = lax.axis_index("s") * _NC + lax.axis_index("c")
        pltpu.sync_copy(table_hbm, table_v)
        lane10 = lax.iota(jnp.int32, _L) * _W

        def chunk_body(c, carry):
            base = wid * n_per_w + c * _CHUNK
            pltpu.sync_copy(idx_hbm.at[pl.ds(base, _CHUNK)], idx_v)

            def vec_body(j, carry2):
                idx16 = idx_v[pl.ds(j * _L, _L)]
                src = idx16 * _W
                dst = j * (_L * _W) + lane10
                for w in range(_W):
                    val = plsc.load_gather(table_v, [src + w])
                    plsc.store_scatter(out_v, [dst + w], val)
                return carry2

            lax.fori_loop(0, _CHUNK // _L, vec_body, 0, unroll=4)
            pltpu.sync_copy(out_v, out_hbm.at[pl.ds(base * _W, _CHUNK * _W)])
            return carry

        lax.fori_loop(0, n_chunks, chunk_body, 0)

    return body


def kernel(x, table):
    b, s = x.shape
    rows, d = table.shape
    n_idx = b * s
    idx_flat = x.reshape(n_idx).astype(jnp.int32)
    table_words = lax.bitcast_convert_type(
        table.reshape(rows * d // 2, 2), jnp.int32
    )
    out_words = _sc_lookup(n_idx)(table_words, idx_flat)
    out = lax.bitcast_convert_type(out_words, jnp.float16)
    return out.reshape(b, s, d)


# speedup vs baseline: 4.5295x; 1.0948x over previous
import functools

import jax
import jax.numpy as jnp
from jax import lax
from jax.experimental import pallas as pl
from jax.experimental.pallas import tpu as pltpu
from jax.experimental.pallas import tpu_sc as plsc

_info = plsc.get_sparse_core_info()
_NC = _info.num_cores
_NS = _info.num_subcores
_NW = _NC * _NS
_L = _info.num_lanes

_ROWS = 10
_W = 10
_CHUNK = 3200


def _sc_lookup(n_idx: int):
    n_per_w = n_idx // _NW
    n_chunks = n_per_w // _CHUNK
    assert n_per_w % _CHUNK == 0 and _CHUNK % _L == 0 and n_chunks % 2 == 0

    mesh = plsc.VectorSubcoreMesh(core_axis_name="c", subcore_axis_name="s")

    @functools.partial(
        pl.kernel,
        mesh=mesh,
        out_type=jax.ShapeDtypeStruct((n_idx * _W,), jnp.int32),
        compiler_params=pltpu.CompilerParams(
            needs_layout_passes=False,
            disable_bounds_checks=True,
        ),
        scratch_types=[
            pltpu.VMEM((_ROWS * _W,), jnp.int32),
            pltpu.VMEM((_CHUNK,), jnp.int32),
            pltpu.VMEM((_CHUNK,), jnp.int32),
            pltpu.VMEM((_CHUNK * _W,), jnp.int32),
            pltpu.VMEM((_CHUNK * _W,), jnp.int32),
            pltpu.SemaphoreType.DMA,
            pltpu.SemaphoreType.DMA,
            pltpu.SemaphoreType.DMA,
            pltpu.SemaphoreType.DMA,
        ],
    )
    def body(table_hbm, idx_hbm, out_hbm, table_v,
             idx_v0, idx_v1, out_v0, out_v1,
             in_sem0, in_sem1, out_sem0, out_sem1):
        wid = lax.axis_index("s") * _NC + lax.axis_index("c")
        pltpu.sync_copy(table_hbm, table_v)
        lane10 = lax.iota(jnp.int32, _L) * _W
        idx_bufs = (idx_v0, idx_v1)
        out_bufs = (out_v0, out_v1)
        in_sems = (in_sem0, in_sem1)
        out_sems = (out_sem0, out_sem1)

        def idx_src(c):
            return idx_hbm.at[pl.ds(wid * n_per_w + c * _CHUNK, _CHUNK)]

        def out_dst(c):
            return out_hbm.at[
                pl.ds((wid * n_per_w + c * _CHUNK) * _W, _CHUNK * _W)
            ]

        pltpu.async_copy(idx_src(0), idx_bufs[0], in_sems[0])

        def chunk_pair(c2, carry):
            for b in range(2):
                c = c2 * 2 + b
                idx_v = idx_bufs[b]
                out_v = out_bufs[b]
                pltpu.make_async_copy(idx_src(c), idx_v, in_sems[b]).wait()

                @pl.when(c + 1 < n_chunks)
                def _():
                    pltpu.async_copy(
                        idx_src(c + 1), idx_bufs[1 - b], in_sems[1 - b]
                    )

                @pl.when(c >= 2)
                def _():
                    pltpu.make_async_copy(
                        out_v, out_dst(c - 2), out_sems[b]
                    ).wait()

                def vec_body(j):
                    idx16 = idx_v[pl.ds(j * _L, _L)]
                    src = idx16 * _W
                    dst = j * (_L * _W) + lane10
                    for w in range(_W):
                        val = plsc.load_gather(table_v, [src + w])
                        plsc.store_scatter(out_v, [dst + w], val)

                plsc.parallel_loop(0, _CHUNK // _L, unroll=8)(vec_body)
                pltpu.async_copy(out_v, out_dst(c), out_sems[b])
            return carry

        lax.fori_loop(0, n_chunks // 2, chunk_pair, 0)
        pltpu.make_async_copy(out_bufs[0], out_dst(n_chunks - 2),
                              out_sems[0]).wait()
        pltpu.make_async_copy(out_bufs[1], out_dst(n_chunks - 1),
                              out_sems[1]).wait()

    return body


def kernel(x, table):
    b, s = x.shape
    rows, d = table.shape
    n_idx = b * s
    idx_flat = x.reshape(n_idx).astype(jnp.int32)
    table_words = lax.bitcast_convert_type(
        table.reshape(rows * d // 2, 2), jnp.int32
    )
    out_words = _sc_lookup(n_idx)(table_words, idx_flat)
    out = lax.bitcast_convert_type(out_words, jnp.float16)
    return out.reshape(b, s, d)


# speedup vs baseline: 23.1498x; 5.1109x over previous
import functools

import jax
import jax.numpy as jnp
from jax import lax
from jax.experimental import pallas as pl
from jax.experimental.pallas import tpu as pltpu
from jax.experimental.pallas import tpu_sc as plsc

_info = plsc.get_sparse_core_info()
_NC = _info.num_cores
_NS = _info.num_subcores
_NW = _NC * _NS
_L = _info.num_lanes

_ROWS = 10
_D = 20
_W = 10
_PT = _ROWS * _ROWS * _D
_BT = 128
_DC = 2


def _sc_lookup(b: int, s: int):
    n_ps = s // 2
    n_blk = b // (_NW * _BT)
    n_dc = _D // _DC
    assert b % (_NW * _BT) == 0 and s % 2 == 0

    mesh = plsc.VectorSubcoreMesh(core_axis_name="c", subcore_axis_name="s")

    @functools.partial(
        pl.kernel,
        mesh=mesh,
        out_type=jax.ShapeDtypeStruct((2 * _D * n_ps, b), jnp.float16),
        compiler_params=pltpu.CompilerParams(
            needs_layout_passes=False,
            disable_bounds_checks=True,
        ),
        scratch_types=[
            pltpu.VMEM((_ROWS * _W,), jnp.int32),
            pltpu.VMEM((_PT,), jnp.int32),
            pltpu.VMEM((s, _BT), jnp.int32),
            pltpu.VMEM((s, _BT), jnp.int32),
            pltpu.VMEM((n_ps, _BT), jnp.int32),
            pltpu.VMEM((_DC * n_ps, _BT), jnp.int32),
            pltpu.VMEM((_DC * n_ps, _BT), jnp.int32),
            pltpu.SemaphoreType.DMA,
            pltpu.SemaphoreType.DMA,
            pltpu.SemaphoreType.DMA,
            pltpu.SemaphoreType.DMA,
        ],
    )
    def body(table_hbm, idxt_hbm, out_hbm, table_v, ptab_v,
             idx_v0, idx_v1, pidx_v, out_v0, out_v1,
             in_sem0, in_sem1, out_sem0, out_sem1):
        wid = lax.axis_index("s") * _NC + lax.axis_index("c")
        out_words = out_hbm.bitcast(jnp.int32)
        pltpu.sync_copy(table_hbm, table_v)

        lane = lax.iota(jnp.int32, _L)

        def ptab_body(j):
            pos = j * _L + lane
            pidx = pos // _D
            d = pos - pidx * _D
            i0 = pidx // _ROWS
            i1 = pidx - i0 * _ROWS
            w = d // 2
            sh = (d - w * 2) * 16
            w0 = plsc.load_gather(table_v, [i0 * _W + w])
            w1 = plsc.load_gather(table_v, [i1 * _W + w])
            v0 = lax.shift_right_logical(w0, sh) & 0xFFFF
            v1 = lax.shift_right_logical(w1, sh) & 0xFFFF
            plsc.store_scatter(ptab_v, [pos], v0 | lax.shift_left(v1, 16))

        plsc.parallel_loop(0, _PT // _L, unroll=4)(ptab_body)

        idx_bufs = (idx_v0, idx_v1)
        out_bufs = (out_v0, out_v1)
        in_sems = (in_sem0, in_sem1)
        out_sems = (out_sem0, out_sem1)

        def idx_src(k):
            return idxt_hbm.at[:, pl.ds(wid * (n_blk * _BT) + k * _BT, _BT)]

        def out_dst(k, dc):
            return out_words.at[
                pl.ds(dc * (_DC * n_ps), _DC * n_ps),
                pl.ds(wid * (n_blk * _BT) + k * _BT, _BT),
            ]

        pltpu.async_copy(idx_src(0), idx_bufs[0], in_sems[0])

        for k in range(n_blk):
            idx_v = idx_bufs[k % 2]
            pltpu.make_async_copy(idx_src(k), idx_v, in_sems[k % 2]).wait()
            if k + 1 < n_blk:
                pltpu.async_copy(
                    idx_src(k + 1), idx_bufs[(k + 1) % 2], in_sems[(k + 1) % 2]
                )

            def pidx_body(ps, carry):
                for l in range(_BT // _L):
                    e16 = idx_v[2 * ps, pl.ds(l * _L, _L)]
                    o16 = idx_v[2 * ps + 1, pl.ds(l * _L, _L)]
                    pidx_v[ps, pl.ds(l * _L, _L)] = (e16 * _ROWS + o16) * _D
                return carry

            lax.fori_loop(0, n_ps, pidx_body, 0, unroll=4)

            for dc in range(n_dc):
                q = k * n_dc + dc
                out_v = out_bufs[q % 2]
                if q >= 2:
                    prev_k, prev_dc = divmod(q - 2, n_dc)
                    pltpu.make_async_copy(
                        out_v, out_dst(prev_k, prev_dc), out_sems[q % 2]
                    ).wait()

                def gat_body(ps, carry):
                    for l in range(_BT // _L):
                        p20 = pidx_v[ps, pl.ds(l * _L, _L)]
                        for dd in range(_DC):
                            val = plsc.load_gather(
                                ptab_v, [p20 + (dc * _DC + dd)]
                            )
                            out_v[dd * n_ps + ps, pl.ds(l * _L, _L)] = val
                    return carry

                lax.fori_loop(0, n_ps, gat_body, 0, unroll=2)
                pltpu.async_copy(out_v, out_dst(k, dc), out_sems[q % 2])

        last = n_blk * n_dc
        for q in (last - 2, last - 1):
            fk, fdc = divmod(q, n_dc)
            pltpu.make_async_copy(
                out_bufs[q % 2], out_dst(fk, fdc), out_sems[q % 2]
            ).wait()

    return body


def kernel(x, table):
    b, s = x.shape
    rows, d = table.shape
    idx_t = x.T.astype(jnp.int32)
    table_words = lax.bitcast_convert_type(
        table.reshape(rows * d // 2, 2), jnp.int32
    )
    y = _sc_lookup(b, s)(table_words, idx_t)
    return y.reshape(d, s, b).transpose(2, 1, 0)


# speedup vs baseline: 91.8652x; 3.9683x over previous
import functools

import jax
import jax.numpy as jnp
from jax import lax
from jax.experimental import pallas as pl
from jax.experimental.pallas import tpu as pltpu
from jax.experimental.pallas import tpu_sc as plsc

_info = plsc.get_sparse_core_info()
_NC = _info.num_cores
_NS = _info.num_subcores
_NW = _NC * _NS
_L = _info.num_lanes

_ROWS = 10
_D = 20
_W = 10
_PT = _ROWS * _ROWS * _D
_BT = 128
_DC = 2


def _sc_lookup(b: int, s: int):
    n_ps = s // 2
    n_blk = b // (_NW * _BT)
    n_dc = _D // _DC
    assert b % (_NW * _BT) == 0 and s % 2 == 0

    mesh = plsc.VectorSubcoreMesh(core_axis_name="c", subcore_axis_name="s")

    @functools.partial(
        pl.kernel,
        mesh=mesh,
        out_type=jax.ShapeDtypeStruct((2 * _D * n_ps, b), jnp.float16),
        compiler_params=pltpu.CompilerParams(
            needs_layout_passes=False,
            disable_bounds_checks=True,
        ),
        scratch_types=[
            pltpu.VMEM((_ROWS * _W,), jnp.int32),
            pltpu.VMEM((_PT,), jnp.int32),
            pltpu.VMEM((s, _BT), jnp.int32),
            pltpu.VMEM((s, _BT), jnp.int32),
            pltpu.VMEM((n_ps, _BT), jnp.int32),
            pltpu.VMEM((_DC * n_ps, _BT), jnp.int32),
            pltpu.VMEM((_DC * n_ps, _BT), jnp.int32),
            pltpu.SemaphoreType.DMA,
            pltpu.SemaphoreType.DMA,
            pltpu.SemaphoreType.DMA,
            pltpu.SemaphoreType.DMA,
        ],
    )
    def body(table_hbm, idxt_hbm, out_hbm, table_v, ptab_v,
             idx_v0, idx_v1, pidx_v, out_v0, out_v1,
             in_sem0, in_sem1, out_sem0, out_sem1):
        wid = lax.axis_index("s") * _NC + lax.axis_index("c")
        out_words = out_hbm.bitcast(jnp.int32)
        pltpu.sync_copy(table_hbm, table_v)

        lane = lax.iota(jnp.int32, _L)

        def ptab_body(j):
            pos = j * _L + lane
            pidx = pos // _D
            d = pos - pidx * _D
            i0 = pidx // _ROWS
            i1 = pidx - i0 * _ROWS
            w = d // 2
            sh = (d - w * 2) * 16
            w0 = plsc.load_gather(table_v, [i0 * _W + w])
            w1 = plsc.load_gather(table_v, [i1 * _W + w])
            v0 = lax.shift_right_logical(w0, sh) & 0xFFFF
            v1 = lax.shift_right_logical(w1, sh) & 0xFFFF
            plsc.store_scatter(ptab_v, [pos], v0 | lax.shift_left(v1, 16))

        plsc.parallel_loop(0, _PT // _L, unroll=4)(ptab_body)

        idx_bufs = (idx_v0, idx_v1)
        out_bufs = (out_v0, out_v1)
        in_sems = (in_sem0, in_sem1)
        out_sems = (out_sem0, out_sem1)

        def idx_src(k):
            return idxt_hbm.at[:, pl.ds(wid * (n_blk * _BT) + k * _BT, _BT)]

        def out_dst(k, dc):
            return out_words.at[
                pl.ds(dc * (_DC * n_ps), _DC * n_ps),
                pl.ds(wid * (n_blk * _BT) + k * _BT, _BT),
            ]

        pltpu.async_copy(idx_src(0), idx_bufs[0], in_sems[0])

        def k_pair(k2, carry):
            for kk in range(2):
                k = k2 * 2 + kk
                idx_v = idx_bufs[kk]
                pltpu.make_async_copy(idx_src(k), idx_v, in_sems[kk]).wait()

                @pl.when(k + 1 < n_blk)
                def _():
                    pltpu.async_copy(
                        idx_src(k + 1), idx_bufs[1 - kk], in_sems[1 - kk]
                    )

                def pidx_body(ps):
                    for l in range(_BT // _L):
                        e16 = idx_v[2 * ps, pl.ds(l * _L, _L)]
                        o16 = idx_v[2 * ps + 1, pl.ds(l * _L, _L)]
                        pidx_v[ps, pl.ds(l * _L, _L)] = (e16 * _ROWS + o16) * _D

                plsc.parallel_loop(0, n_ps, unroll=4)(pidx_body)

                def dc_pair(dc2, carry2):
                    for dcc in range(2):
                        dc = dc2 * 2 + dcc
                        q = k * n_dc + dc
                        out_v = out_bufs[dcc]

                        @pl.when(q >= 2)
                        def _():
                            pltpu.make_async_copy(
                                out_v, out_dst(0, 0), out_sems[dcc]
                            ).wait()

                        pd = dc * _DC

                        def gat_body(ps):
                            for l in range(_BT // _L):
                                p20 = pidx_v[ps, pl.ds(l * _L, _L)] + pd
                                for dd in range(_DC):
                                    val = plsc.load_gather(ptab_v, [p20 + dd])
                                    out_v[dd * n_ps + ps,
                                          pl.ds(l * _L, _L)] = val

                        plsc.parallel_loop(0, n_ps, unroll=2)(gat_body)
                        pltpu.async_copy(out_v, out_dst(k, dc), out_sems[dcc])
                    return carry2

                lax.fori_loop(0, n_dc // 2, dc_pair, 0)
            return carry

        lax.fori_loop(0, n_blk // 2, k_pair, 0)
        for q in (n_blk * n_dc - 2, n_blk * n_dc - 1):
            pltpu.make_async_copy(
                out_bufs[q % 2], out_dst(n_blk - 1, q % n_dc), out_sems[q % 2]
            ).wait()

    return body


def kernel(x, table):
    b, s = x.shape
    rows, d = table.shape
    idx_t = x.T.astype(jnp.int32)
    table_words = lax.bitcast_convert_type(
        table.reshape(rows * d // 2, 2), jnp.int32
    )
    y = _sc_lookup(b, s)(table_words, idx_t)
    return y.reshape(d, s, b).transpose(2, 1, 0)


# speedup vs baseline: 105.8896x; 1.1527x over previous
import functools

import jax
import jax.numpy as jnp
from jax import lax
from jax.experimental import pallas as pl
from jax.experimental.pallas import tpu as pltpu
from jax.experimental.pallas import tpu_sc as plsc

_info = plsc.get_sparse_core_info()
_NC = _info.num_cores
_NS = _info.num_subcores
_NW = _NC * _NS
_L = _info.num_lanes

_ROWS = 10
_D = 20
_W = 10
_PT = _ROWS * _ROWS * _D
_BT = 128
_DC = 4


def _sc_lookup(b: int, s: int):
    n_ps = s // 2
    n_blk = b // (_NW * _BT)
    n_dc = _D // _DC
    assert b % (_NW * _BT) == 0 and s % 2 == 0

    mesh = plsc.VectorSubcoreMesh(core_axis_name="c", subcore_axis_name="s")

    @functools.partial(
        pl.kernel,
        mesh=mesh,
        out_type=jax.ShapeDtypeStruct((2 * _D * n_ps, b), jnp.float16),
        compiler_params=pltpu.CompilerParams(
            needs_layout_passes=False,
            disable_bounds_checks=True,
        ),
        scratch_types=[
            pltpu.VMEM((_ROWS * _W,), jnp.int32),
            pltpu.VMEM((_PT,), jnp.int32),
            pltpu.VMEM((s, _BT), jnp.int32),
            pltpu.VMEM((_DC * n_ps, _BT), jnp.int32),
            pltpu.VMEM((_DC * n_ps, _BT), jnp.int32),
            pltpu.SemaphoreType.DMA,
            pltpu.SemaphoreType.DMA,
        ],
    )
    def body(table_hbm, idxt_hbm, out_hbm, table_v, ptab_v,
             idx_v, out_v0, out_v1, out_sem0, out_sem1):
        wid = lax.axis_index("s") * _NC + lax.axis_index("c")
        out_words = out_hbm.bitcast(jnp.int32)
        pltpu.sync_copy(table_hbm, table_v)

        lane = lax.iota(jnp.int32, _L)

        def ptab_body(j):
            pos = j * _L + lane
            pidx = pos // _D
            d = pos - pidx * _D
            i0 = pidx // _ROWS
            i1 = pidx - i0 * _ROWS
            w = d // 2
            sh = (d - w * 2) * 16
            w0 = plsc.load_gather(table_v, [i0 * _W + w])
            w1 = plsc.load_gather(table_v, [i1 * _W + w])
            v0 = lax.shift_right_logical(w0, sh) & 0xFFFF
            v1 = lax.shift_right_logical(w1, sh) & 0xFFFF
            plsc.store_scatter(ptab_v, [pos], v0 | lax.shift_left(v1, 16))

        plsc.parallel_loop(0, _PT // _L, unroll=4)(ptab_body)

        out_bufs = (out_v0, out_v1)
        out_sems = (out_sem0, out_sem1)

        def idx_src(k):
            return idxt_hbm.at[:, pl.ds(wid * (n_blk * _BT) + k * _BT, _BT)]

        def out_dst(k, dc):
            return out_words.at[
                pl.ds(dc * (_DC * n_ps), _DC * n_ps),
                pl.ds(wid * (n_blk * _BT) + k * _BT, _BT),
            ]

        n_q = n_blk * n_dc

        def q_pair(q2, carry):
            for qq in range(2):
                q = q2 * 2 + qq
                k = q // n_dc
                dc = q - k * n_dc
                out_v = out_bufs[qq]

                @pl.when(dc == 0)
                def _():
                    pltpu.sync_copy(idx_src(k), idx_v)

                    def pidx_body(ps):
                        for l in range(_BT // _L):
                            e16 = idx_v[2 * ps, pl.ds(l * _L, _L)]
                            o16 = idx_v[2 * ps + 1, pl.ds(l * _L, _L)]
                            idx_v[2 * ps, pl.ds(l * _L, _L)] = (
                                (e16 * _ROWS + o16) * _D
                            )

                    plsc.parallel_loop(0, n_ps, unroll=4)(pidx_body)

                @pl.when(q >= 2)
                def _():
                    pltpu.make_async_copy(
                        out_v, out_dst(0, 0), out_sems[qq]
                    ).wait()

                pd = dc * _DC

                def gat_body(ps):
                    for l in range(_BT // _L):
                        p20 = idx_v[2 * ps, pl.ds(l * _L, _L)] + pd
                        for dd in range(_DC):
                            val = plsc.load_gather(ptab_v, [p20 + dd])
                            out_v[dd * n_ps + ps, pl.ds(l * _L, _L)] = val

                plsc.parallel_loop(0, n_ps, unroll=2)(gat_body)
                pltpu.async_copy(out_v, out_dst(k, dc), out_sems[qq])
            return carry

        lax.fori_loop(0, n_q // 2, q_pair, 0)
        for q in (n_q - 2, n_q - 1):
            pltpu.make_async_copy(
                out_bufs[q % 2], out_dst(0, 0), out_sems[q % 2]
            ).wait()

    return body


def kernel(x, table):
    b, s = x.shape
    rows, d = table.shape
    idx_t = x.T.astype(jnp.int32)
    table_words = lax.bitcast_convert_type(
        table.reshape(rows * d // 2, 2), jnp.int32
    )
    y = _sc_lookup(b, s)(table_words, idx_t)
    return y.reshape(d, s, b).transpose(2, 1, 0)
